# baseline (speedup 1.0000x reference)
import jax
import jax.numpy as jnp
from jax.experimental import pallas as pl


def _apply_body(img_ref, w_ref, p_ref, out_ref):
    mw = w_ref[...] * (1.0 - p_ref[...])
    out_ref[...] = img_ref[...] * mw[None, :, :]


def kernel(image, weight, pruned):
    B, C, H, W = image.shape
    img = image.reshape(B * C, H, W)
    pf = pruned.astype(jnp.float32)
    K = 4
    out = pl.pallas_call(
        _apply_body,
        grid=(B * C // K,),
        in_specs=[
            pl.BlockSpec((K, H, W), lambda i: (i, 0, 0)),
            pl.BlockSpec((H, W), lambda i: (0, 0)),
            pl.BlockSpec((H, W), lambda i: (0, 0)),
        ],
        out_specs=pl.BlockSpec((K, H, W), lambda i: (i, 0, 0)),
        out_shape=jax.ShapeDtypeStruct((B * C, H, W), jnp.float32),
    )(img, weight, pf)
    return out.reshape(1, B, C, H, W)


# speedup vs baseline: 1.0225x; 1.0225x over previous
import jax
import jax.numpy as jnp
from jax.experimental import pallas as pl
from jax.experimental.pallas import tpu as pltpu


def _apply_body(img_ref, w_ref, p_ref, out_ref, mw_ref):
    @pl.when(pl.program_id(0) == 0)
    def _():
        mw_ref[...] = jnp.where(p_ref[...], 0.0, w_ref[...])

    out_ref[...] = img_ref[...] * mw_ref[...][None, :, :]


def kernel(image, weight, pruned):
    B, C, H, W = image.shape
    img = image.reshape(B * C, H, W)
    K = 8
    out = pl.pallas_call(
        _apply_body,
        grid=(B * C // K,),
        in_specs=[
            pl.BlockSpec((K, H, W), lambda i: (i, 0, 0)),
            pl.BlockSpec((H, W), lambda i: (0, 0)),
            pl.BlockSpec((H, W), lambda i: (0, 0)),
        ],
        out_specs=pl.BlockSpec((K, H, W), lambda i: (i, 0, 0)),
        out_shape=jax.ShapeDtypeStruct((B * C, H, W), jnp.float32),
        scratch_shapes=[pltpu.VMEM((H, W), jnp.float32)],
    )(img, weight, pruned)
    return out.reshape(1, B, C, H, W)
